# baseline (speedup 1.0000x reference)
import functools

import jax
import jax.numpy as jnp
from jax import lax
from jax.experimental import pallas as pl
from jax.experimental.pallas import tpu as pltpu
from jax.experimental.pallas import tpu_sc as plsc

NC = 2
NS = 16
NW = NC * NS
LN = 16
CK = 128
CKB = 768
CKC = 64
SB = 4
ZR = 64


def _leaky(x, s):
    return jnp.where(x >= 0, x, s * x)


def _prep_body(x_ref, w_in_ref, b_in_ref, w_gat_ref, asw_ref, adw_ref,
               hw_ref, asrc_ref, adst_ref):
    x = x_ref[...]
    h = _leaky(
        lax.dot_general(x, w_in_ref[...], (((1,), (1,)), ((), ())),
                        preferred_element_type=jnp.float32) + b_in_ref[...],
        0.1)
    wf = w_gat_ref[0]
    hw = lax.dot_general(h, wf, (((1,), (1,)), ((), ())),
                         preferred_element_type=jnp.float32)
    hw_ref[...] = hw.astype(jnp.bfloat16)
    asrc_ref[0] = jnp.sum(hw * asw_ref[0], axis=1, keepdims=True)
    adst_ref[0] = jnp.sum(hw * adw_ref[0], axis=1, keepdims=True)


def _prep(x, w_in, b_in, w_gat, att_src, att_dst, n, d, h, bn):
    nb = n // bn
    return pl.pallas_call(
        _prep_body,
        grid=(h, nb),
        in_specs=[
            pl.BlockSpec((bn, d), lambda f, i: (i, 0)),
            pl.BlockSpec((d, d), lambda f, i: (0, 0)),
            pl.BlockSpec((1, d), lambda f, i: (0, 0)),
            pl.BlockSpec((1, d, d), lambda f, i: (f, 0, 0)),
            pl.BlockSpec((1, 1, d), lambda f, i: (f, 0, 0)),
            pl.BlockSpec((1, 1, d), lambda f, i: (f, 0, 0)),
        ],
        out_specs=[
            pl.BlockSpec((bn, d), lambda f, i: (i, f)),
            pl.BlockSpec((1, bn, 1), lambda f, i: (f, i, 0)),
            pl.BlockSpec((1, bn, 1), lambda f, i: (f, i, 0)),
        ],
        out_shape=[
            jax.ShapeDtypeStruct((n, h * d), jnp.bfloat16),
            jax.ShapeDtypeStruct((h, n, 1), jnp.float32),
            jax.ShapeDtypeStruct((h, n, 1), jnp.float32),
        ],
    )(x, w_in, b_in.reshape(1, d), w_gat.reshape(h, d, d),
      att_src.reshape(h, 1, d), att_dst.reshape(h, 1, d))


def _edge_softmax_denom(asrc, adst, srcp, dstp, n, h, etot, nch):
    ep = NW * CK * nch
    mesh = plsc.VectorSubcoreMesh(core_axis_name="c", subcore_axis_name="s")

    @functools.partial(
        pl.kernel,
        out_type=(
            jax.ShapeDtypeStruct((h, ep), jnp.float32),
            jax.ShapeDtypeStruct((NW, h, n), jnp.float32),
        ),
        mesh=mesh,
        scratch_types=[
            pltpu.VMEM((n,), jnp.float32),
            pltpu.VMEM((n,), jnp.float32),
            pltpu.VMEM((n,), jnp.float32),
            pltpu.VMEM((CKB,), jnp.int32),
            pltpu.VMEM((CKB,), jnp.int32),
            pltpu.VMEM((CKB,), jnp.float32),
        ],
        compiler_params=pltpu.CompilerParams(needs_layout_passes=False),
    )
    def kb(asrc_hbm, adst_hbm, srcp_hbm, dstp_hbm, ex_hbm, dpart_hbm,
           as_v, ad_v, den_v, src_v, dst_v, ex_v):
        cid = lax.axis_index("c")
        sid = lax.axis_index("s")
        wid = sid * NC + cid
        base = wid * (nch * CK)
        nchb = (nch * CK) // CKB
        for hh in range(h):
            pltpu.sync_copy(asrc_hbm.at[hh], as_v)
            pltpu.sync_copy(adst_hbm.at[hh], ad_v)

            def zbody(i, _):
                den_v[pl.ds(i * LN, LN)] = jnp.zeros((LN,), jnp.float32)
                return 0

            lax.fori_loop(0, n // LN, zbody, 0)

            def chunk(ci, _):
                eb = base + ci * CKB
                pltpu.sync_copy(srcp_hbm.at[pl.ds(eb, CKB)], src_v)
                pltpu.sync_copy(dstp_hbm.at[pl.ds(eb, CKB)], dst_v)

                def group(g, _):
                    sl = pl.ds(g * LN, LN)
                    si = src_v[sl]
                    di = dst_v[sl]
                    a = plsc.load_gather(as_v, [si]) + plsc.load_gather(ad_v, [di])
                    e = jnp.exp(_leaky(a, 0.2))
                    eidx = eb + g * LN + lax.iota(jnp.int32, LN)
                    e = jnp.where(eidx < etot, e, 0.0)
                    ex_v[sl] = e
                    plsc.addupdate_scatter(den_v, [di], e)
                    return 0

                lax.fori_loop(0, CKB // LN, group, 0)
                pltpu.sync_copy(ex_v, ex_hbm.at[hh, pl.ds(eb, CKB)])
                return 0

            lax.fori_loop(0, nchb, chunk, 0)
            pltpu.sync_copy(den_v, dpart_hbm.at[wid, hh])

    return kb(asrc, adst, srcp, dstp)


def _rinv_body(dpart_ref, rinv_ref):
    rinv_ref[...] = 1.0 / jnp.sum(dpart_ref[...], axis=0)


def _rinv(dpart, n, h):
    return pl.pallas_call(
        _rinv_body,
        out_shape=jax.ShapeDtypeStruct((h, n), jnp.float32),
    )(dpart)


def _edge_attn(rinv, ex, dstp, n, h, nch):
    ep = NW * CK * nch
    mesh = plsc.VectorSubcoreMesh(core_axis_name="c", subcore_axis_name="s")

    @functools.partial(
        pl.kernel,
        out_type=jax.ShapeDtypeStruct((h, ep), jnp.float32),
        mesh=mesh,
        scratch_types=[
            pltpu.VMEM((n,), jnp.float32),
            pltpu.VMEM((CKB,), jnp.int32),
            pltpu.VMEM((CKB,), jnp.float32),
            pltpu.VMEM((CKB,), jnp.float32),
        ],
        compiler_params=pltpu.CompilerParams(needs_layout_passes=False),
    )
    def kb2(rinv_hbm, ex_hbm, dstp_hbm, attn_hbm, rinv_v, dst_v, ex_v, at_v):
        cid = lax.axis_index("c")
        sid = lax.axis_index("s")
        wid = sid * NC + cid
        base = wid * (nch * CK)
        nchb = (nch * CK) // CKB
        for hh in range(h):
            pltpu.sync_copy(rinv_hbm.at[hh], rinv_v)

            def chunk(ci, _):
                eb = base + ci * CKB
                pltpu.sync_copy(dstp_hbm.at[pl.ds(eb, CKB)], dst_v)
                pltpu.sync_copy(ex_hbm.at[hh, pl.ds(eb, CKB)], ex_v)

                def group(g, _):
                    sl = pl.ds(g * LN, LN)
                    at_v[sl] = ex_v[sl] * plsc.load_gather(rinv_v, [dst_v[sl]])
                    return 0

                lax.fori_loop(0, CKB // LN, group, 0)
                pltpu.sync_copy(at_v, attn_hbm.at[hh, pl.ds(eb, CKB)])
                return 0

            lax.fori_loop(0, nchb, chunk, 0)

    return kb2(rinv, ex, dstp)


def _edge_aggregate(hw2, attn, srcp2, dstp2, n, d, h, nch, npad):
    nzc = npad // ZR
    hd = h * d
    scw = SB * CKC
    nsc = (nch * CK) // scw
    mesh = plsc.VectorSubcoreMesh(core_axis_name="c", subcore_axis_name="s")

    @functools.partial(
        pl.kernel,
        out_type=jax.ShapeDtypeStruct((NC, npad, d), jnp.float32),
        mesh=mesh,
        scratch_types=[
            pltpu.VMEM((scw,), jnp.int32),
            pltpu.VMEM((scw,), jnp.int32),
            pltpu.VMEM((h, scw), jnp.float32),
            pltpu.VMEM((CKC, hd // 2), jnp.int32),
            pltpu.VMEM((CKC, hd // 2), jnp.int32),
            pltpu.VMEM((CKC, d), jnp.float32),
            pltpu.VMEM((CKC, d), jnp.float32),
            pltpu.VMEM_SHARED((npad, d), jnp.float32),
            pltpu.SemaphoreType.DMA,
            pltpu.SemaphoreType.DMA,
            pltpu.SemaphoreType.DMA,
            pltpu.SemaphoreType.DMA,
        ],
        compiler_params=pltpu.CompilerParams(needs_layout_passes=False),
    )
    def kc(hw_hbm, attn_hbm, srcp_hbm, dstp_hbm, out_hbm,
           src_sb, dst_sb, at_v, grow_a, grow_b, orow_a, orow_b, acc_sh,
           sem_a, sem_b, ssem_a, ssem_b):
        cid = lax.axis_index("c")
        sid = lax.axis_index("s")
        wid = sid * NC + cid
        base = wid * (nsc * scw)

        def zrow(k, _):
            for j in range(d // LN):
                orow_a[k, pl.ds(j * LN, LN)] = jnp.zeros((LN,), jnp.float32)
            return 0

        lax.fori_loop(0, ZR, zrow, 0)

        def zacc(ch, _):
            @pl.when(lax.rem(ch, NS) == sid)
            def _():
                pltpu.sync_copy(orow_a, acc_sh.at[pl.ds(ch * ZR, ZR)])
            return 0

        lax.fori_loop(0, nzc, zacc, 0)
        plsc.subcore_barrier()

        bufs = (grow_a, grow_b)
        sems = (sem_a, sem_b)
        orows = (orow_a, orow_b)
        ssems = (ssem_a, ssem_b)

        def schunk(si, _):
            sbase = base + si * scw
            pltpu.sync_copy(srcp_hbm.at[pl.ds(sbase, scw)], src_sb)
            pltpu.sync_copy(dstp_hbm.at[pl.ds(sbase, scw)], dst_sb)
            for hh in range(h):
                pltpu.sync_copy(attn_hbm.at[hh, pl.ds(sbase, scw)],
                                at_v.at[hh])
            cps = [None, None]
            scs = [None, None]
            cps[0] = pltpu.async_copy(
                hw_hbm.at[src_sb.at[pl.ds(0, CKC)]], bufs[0], sems[0])
            for b in range(SB):
                p = b % 2
                cps[p].wait()
                if b + 1 < SB:
                    q = (b + 1) % 2
                    cps[q] = pltpu.async_copy(
                        hw_hbm.at[src_sb.at[pl.ds((b + 1) * CKC, CKC)]],
                        bufs[q], sems[q])
                buf = bufs[p]
                orow = orows[p]
                if scs[p] is not None:
                    scs[p].wait()

                def scale(k, _):
                    kv = (jnp.broadcast_to(k, (LN,)).astype(jnp.int32)
                          + b * CKC)
                    avs = [plsc.load_gather(
                        at_v, [jnp.full((LN,), hh, jnp.int32), kv])
                        for hh in range(h)]
                    for jj in range(d // (2 * LN)):
                        ta = tb = None
                        for hh in range(h):
                            u = plsc.bitcast(
                                buf[k, pl.ds(hh * (d // 2) + jj * LN, LN)],
                                jnp.bfloat16)
                            ea, eb = plsc.unpack(
                                u, format=plsc.PackFormat.INTERLEAVED)
                            if ta is None:
                                ta = ea * avs[hh]
                                tb = eb * avs[hh]
                            else:
                                ta = ta + ea * avs[hh]
                                tb = tb + eb * avs[hh]
                        orow[k, pl.ds(jj * 2 * LN, LN)] = ta
                        orow[k, pl.ds(jj * 2 * LN + LN, LN)] = tb
                    return 0

                lax.fori_loop(0, CKC, scale, 0)
                scs[p] = pltpu.async_copy(
                    orow, acc_sh.at[dst_sb.at[pl.ds(b * CKC, CKC)]],
                    ssems[p], add=True)
            for p in range(2):
                if scs[p] is not None:
                    scs[p].wait()
            return 0

        lax.fori_loop(0, nsc, schunk, 0)

        plsc.subcore_barrier()

        def wout(ch, _):
            @pl.when(lax.rem(ch, NS) == sid)
            def _():
                pltpu.sync_copy(acc_sh.at[pl.ds(ch * ZR, ZR)],
                                out_hbm.at[cid, pl.ds(ch * ZR, ZR)])
            return 0

        lax.fori_loop(0, nzc, wout, 0)

    return kc(hw2, attn, srcp2, dstp2)


def _node_body(parts_ref, b_gat_ref, wg1_ref, bg1_ref, wg2_ref, bg2_ref,
               h2_ref, g_ref):
    p = (parts_ref[0] + parts_ref[1]) * 0.25
    h2 = _leaky(p + b_gat_ref[...], 0.1)
    h2_ref[...] = h2
    u = jnp.maximum(
        lax.dot_general(h2, wg1_ref[...], (((1,), (1,)), ((), ())),
                        preferred_element_type=jnp.float32) + bg1_ref[...],
        0.0)
    g_ref[...] = jnp.sum(u * wg2_ref[...], axis=1, keepdims=True) + bg2_ref[0, 0]


def _node_stage(parts, b_gat, wg1, bg1, wg2, bg2, n, d, bn, npad):
    nb = n // bn
    return pl.pallas_call(
        _node_body,
        grid=(nb,),
        in_specs=[
            pl.BlockSpec((NC, bn, d), lambda i: (0, i, 0)),
            pl.BlockSpec((1, d), lambda i: (0, 0)),
            pl.BlockSpec((d, d), lambda i: (0, 0)),
            pl.BlockSpec((1, d), lambda i: (0, 0)),
            pl.BlockSpec((1, d), lambda i: (0, 0)),
            pl.BlockSpec((1, 1), lambda i: (0, 0)),
        ],
        out_specs=[
            pl.BlockSpec((bn, d), lambda i: (i, 0)),
            pl.BlockSpec((bn, 1), lambda i: (i, 0)),
        ],
        out_shape=[
            jax.ShapeDtypeStruct((n, d), jnp.float32),
            jax.ShapeDtypeStruct((n, 1), jnp.float32),
        ],
    )(parts, b_gat.reshape(1, d), wg1, bg1.reshape(1, d), wg2.reshape(1, d),
      bg2.reshape(1, 1))


def _pool_body(h2_ref, g_ref, batch_ref, w_out_ref, b_out_ref, out_ref,
               *, num_graphs):
    gv = g_ref[...]
    seg = batch_ref[...]
    n = gv.shape[1]
    ids = lax.broadcasted_iota(jnp.int32, (num_graphs, n), 0)
    cmp = ids == seg
    cmpf = cmp.astype(jnp.float32)
    gb = jnp.where(cmp, jnp.broadcast_to(gv, (num_graphs, n)), -1e30)
    m = jnp.max(gb, axis=1, keepdims=True)
    mn = jnp.sum(cmpf * m, axis=0).reshape(1, n)
    e = jnp.exp(gv - mn)
    s = jnp.sum(cmpf * e, axis=1, keepdims=True)
    sn = jnp.sum(cmpf * s, axis=0).reshape(1, n)
    w = (e / sn).reshape(n, 1)
    h2w = h2_ref[...] * w
    emb = lax.dot_general(cmpf, h2w, (((1,), (0,)), ((), ())),
                          preferred_element_type=jnp.float32)
    out_ref[...] = lax.dot_general(
        emb, w_out_ref[...], (((1,), (1,)), ((), ())),
        preferred_element_type=jnp.float32) + b_out_ref[...]


def _pool(h2, g2, batch2, w_out, b_out, num_graphs, out_dim):
    return pl.pallas_call(
        functools.partial(_pool_body, num_graphs=num_graphs),
        out_shape=jax.ShapeDtypeStruct((num_graphs, out_dim), jnp.float32),
    )(h2, g2, batch2, w_out, b_out.reshape(1, out_dim))


def kernel(x, edge_index, batch, W_in, b_in, W_gat, att_src, att_dst,
           b_gat, Wg1, bg1, Wg2, bg2, W_out, b_out):
    n, d = x.shape
    h = att_src.shape[0]
    out_dim = W_out.shape[0]
    num_graphs = 64
    e = edge_index.shape[1]
    etot = e + n
    nch = -(-etot // (NW * CK))
    ep = NW * CK * nch
    bn = 1000
    npad = -(-n // ZR) * ZR

    loop = jnp.arange(n, dtype=edge_index.dtype)
    pad = jnp.zeros((ep - etot,), edge_index.dtype)
    srcp = jnp.concatenate([edge_index[0], loop, pad])
    dstp = jnp.concatenate([edge_index[1], loop, pad])

    idx = jnp.arange(d)
    grp, r = idx // 32, idx % 32
    perm = 32 * grp + jnp.where(r % 2 == 0, r // 2, 16 + r // 2)
    w_gat_p = W_gat.reshape(h, d, d)[:, perm, :].reshape(h * d, d)
    att_src_p = att_src[:, perm]
    att_dst_p = att_dst[:, perm]

    hw, asrc3, adst3 = _prep(x, W_in, b_in, w_gat_p, att_src_p, att_dst_p,
                             n, d, h, bn)
    asrc = asrc3.reshape(h, n)
    adst = adst3.reshape(h, n)

    ex, dpart = _edge_softmax_denom(asrc, adst, srcp, dstp, n, h, etot, nch)
    rinv = _rinv(dpart, n, h)
    attn = _edge_attn(rinv, ex, dstp, n, h, nch)

    hw_i32 = lax.bitcast_convert_type(
        hw.reshape(n, (h * d) // 2, 2), jnp.int32)
    parts = _edge_aggregate(hw_i32, attn, srcp, dstp, n, d, h, nch, npad)

    h2, g3 = _node_stage(parts, b_gat, Wg1, bg1, Wg2, bg2, n, d, bn, npad)
    g2 = g3.reshape(1, n)
    batch2 = batch.reshape(1, n)

    return _pool(h2, g2, batch2, W_out, b_out, num_graphs, out_dim)


# speedup vs baseline: 1.0217x; 1.0217x over previous
import functools

import jax
import jax.numpy as jnp
from jax import lax
from jax.experimental import pallas as pl
from jax.experimental.pallas import tpu as pltpu
from jax.experimental.pallas import tpu_sc as plsc

NC = 2
NS = 16
NW = NC * NS
LN = 16
CK = 128
CKB = 768
CKC = 32
SB = 12
ZR = 32


def _leaky(x, s):
    return jnp.where(x >= 0, x, s * x)


def _prep_body(x_ref, w_in_ref, b_in_ref, w_gat_ref, asw_ref, adw_ref,
               hw_ref, asrc_ref, adst_ref):
    x = x_ref[...]
    h = _leaky(
        lax.dot_general(x, w_in_ref[...], (((1,), (1,)), ((), ())),
                        preferred_element_type=jnp.float32) + b_in_ref[...],
        0.1)
    wf = w_gat_ref[0]
    hw = lax.dot_general(h, wf, (((1,), (1,)), ((), ())),
                         preferred_element_type=jnp.float32)
    hw_ref[...] = hw.astype(jnp.bfloat16)
    asrc_ref[0] = jnp.sum(hw * asw_ref[0], axis=1, keepdims=True)
    adst_ref[0] = jnp.sum(hw * adw_ref[0], axis=1, keepdims=True)


def _prep(x, w_in, b_in, w_gat, att_src, att_dst, n, d, h, bn):
    nb = n // bn
    return pl.pallas_call(
        _prep_body,
        grid=(h, nb),
        in_specs=[
            pl.BlockSpec((bn, d), lambda f, i: (i, 0)),
            pl.BlockSpec((d, d), lambda f, i: (0, 0)),
            pl.BlockSpec((1, d), lambda f, i: (0, 0)),
            pl.BlockSpec((1, d, d), lambda f, i: (f, 0, 0)),
            pl.BlockSpec((1, 1, d), lambda f, i: (f, 0, 0)),
            pl.BlockSpec((1, 1, d), lambda f, i: (f, 0, 0)),
        ],
        out_specs=[
            pl.BlockSpec((bn, d), lambda f, i: (i, f)),
            pl.BlockSpec((1, bn, 1), lambda f, i: (f, i, 0)),
            pl.BlockSpec((1, bn, 1), lambda f, i: (f, i, 0)),
        ],
        out_shape=[
            jax.ShapeDtypeStruct((n, h * d), jnp.bfloat16),
            jax.ShapeDtypeStruct((h, n, 1), jnp.float32),
            jax.ShapeDtypeStruct((h, n, 1), jnp.float32),
        ],
    )(x, w_in, b_in.reshape(1, d), w_gat.reshape(h, d, d),
      att_src.reshape(h, 1, d), att_dst.reshape(h, 1, d))


def _edge_softmax_denom(asrc, adst, srcp, dstp, n, h, etot, nch):
    ep = NW * CK * nch
    mesh = plsc.VectorSubcoreMesh(core_axis_name="c", subcore_axis_name="s")

    @functools.partial(
        pl.kernel,
        out_type=(
            jax.ShapeDtypeStruct((h, ep), jnp.float32),
            jax.ShapeDtypeStruct((NW, h, n), jnp.float32),
        ),
        mesh=mesh,
        scratch_types=[
            pltpu.VMEM((n,), jnp.float32),
            pltpu.VMEM((n,), jnp.float32),
            pltpu.VMEM((n,), jnp.float32),
            pltpu.VMEM((CKB,), jnp.int32),
            pltpu.VMEM((CKB,), jnp.int32),
            pltpu.VMEM((CKB,), jnp.float32),
        ],
        compiler_params=pltpu.CompilerParams(needs_layout_passes=False),
    )
    def kb(asrc_hbm, adst_hbm, srcp_hbm, dstp_hbm, ex_hbm, dpart_hbm,
           as_v, ad_v, den_v, src_v, dst_v, ex_v):
        cid = lax.axis_index("c")
        sid = lax.axis_index("s")
        wid = sid * NC + cid
        base = wid * (nch * CK)
        nchb = (nch * CK) // CKB
        for hh in range(h):
            pltpu.sync_copy(asrc_hbm.at[hh], as_v)
            pltpu.sync_copy(adst_hbm.at[hh], ad_v)

            def zbody(i, _):
                den_v[pl.ds(i * LN, LN)] = jnp.zeros((LN,), jnp.float32)
                return 0

            lax.fori_loop(0, n // LN, zbody, 0)

            def chunk(ci, _):
                eb = base + ci * CKB
                pltpu.sync_copy(srcp_hbm.at[pl.ds(eb, CKB)], src_v)
                pltpu.sync_copy(dstp_hbm.at[pl.ds(eb, CKB)], dst_v)

                def group(g, _):
                    sl = pl.ds(g * LN, LN)
                    si = src_v[sl]
                    di = dst_v[sl]
                    a = plsc.load_gather(as_v, [si]) + plsc.load_gather(ad_v, [di])
                    e = jnp.exp(_leaky(a, 0.2))
                    eidx = eb + g * LN + lax.iota(jnp.int32, LN)
                    e = jnp.where(eidx < etot, e, 0.0)
                    ex_v[sl] = e
                    plsc.addupdate_scatter(den_v, [di], e)
                    return 0

                lax.fori_loop(0, CKB // LN, group, 0)
                pltpu.sync_copy(ex_v, ex_hbm.at[hh, pl.ds(eb, CKB)])
                return 0

            lax.fori_loop(0, nchb, chunk, 0)
            pltpu.sync_copy(den_v, dpart_hbm.at[wid, hh])

    return kb(asrc, adst, srcp, dstp)


def _rinv_body(dpart_ref, rinv_ref):
    rinv_ref[...] = 1.0 / jnp.sum(dpart_ref[...], axis=0)


def _rinv(dpart, n, h):
    return pl.pallas_call(
        _rinv_body,
        out_shape=jax.ShapeDtypeStruct((h, n), jnp.float32),
    )(dpart)


def _edge_attn(rinv, ex, dstp, n, h, nch):
    ep = NW * CK * nch
    mesh = plsc.VectorSubcoreMesh(core_axis_name="c", subcore_axis_name="s")

    @functools.partial(
        pl.kernel,
        out_type=jax.ShapeDtypeStruct((h, ep), jnp.float32),
        mesh=mesh,
        scratch_types=[
            pltpu.VMEM((n,), jnp.float32),
            pltpu.VMEM((CKB,), jnp.int32),
            pltpu.VMEM((CKB,), jnp.float32),
            pltpu.VMEM((CKB,), jnp.float32),
        ],
        compiler_params=pltpu.CompilerParams(needs_layout_passes=False),
    )
    def kb2(rinv_hbm, ex_hbm, dstp_hbm, attn_hbm, rinv_v, dst_v, ex_v, at_v):
        cid = lax.axis_index("c")
        sid = lax.axis_index("s")
        wid = sid * NC + cid
        base = wid * (nch * CK)
        nchb = (nch * CK) // CKB
        for hh in range(h):
            pltpu.sync_copy(rinv_hbm.at[hh], rinv_v)

            def chunk(ci, _):
                eb = base + ci * CKB
                pltpu.sync_copy(dstp_hbm.at[pl.ds(eb, CKB)], dst_v)
                pltpu.sync_copy(ex_hbm.at[hh, pl.ds(eb, CKB)], ex_v)

                def group(g, _):
                    sl = pl.ds(g * LN, LN)
                    at_v[sl] = ex_v[sl] * plsc.load_gather(rinv_v, [dst_v[sl]])
                    return 0

                lax.fori_loop(0, CKB // LN, group, 0)
                pltpu.sync_copy(at_v, attn_hbm.at[hh, pl.ds(eb, CKB)])
                return 0

            lax.fori_loop(0, nchb, chunk, 0)

    return kb2(rinv, ex, dstp)


def _edge_aggregate(hw2, attn, srcp2, dstp2, n, d, h, nch, npad):
    nzc = npad // ZR
    hd = h * d
    scw = SB * CKC
    nsc = (nch * CK) // scw
    mesh = plsc.VectorSubcoreMesh(core_axis_name="c", subcore_axis_name="s")

    @functools.partial(
        pl.kernel,
        out_type=jax.ShapeDtypeStruct((NC, npad, d), jnp.float32),
        mesh=mesh,
        scratch_types=[
            pltpu.VMEM((scw,), jnp.int32),
            pltpu.VMEM((scw,), jnp.int32),
            pltpu.VMEM((h, scw), jnp.float32),
            pltpu.VMEM((CKC, hd // 2), jnp.int32),
            pltpu.VMEM((CKC, hd // 2), jnp.int32),
            pltpu.VMEM((CKC, d), jnp.float32),
            pltpu.VMEM((CKC, d), jnp.float32),
            pltpu.VMEM_SHARED((npad, d), jnp.float32),
            pltpu.SemaphoreType.DMA,
            pltpu.SemaphoreType.DMA,
            pltpu.SemaphoreType.DMA,
            pltpu.SemaphoreType.DMA,
        ],
        compiler_params=pltpu.CompilerParams(needs_layout_passes=False),
    )
    def kc(hw_hbm, attn_hbm, srcp_hbm, dstp_hbm, out_hbm,
           src_sb, dst_sb, at_v, grow_a, grow_b, orow_a, orow_b, acc_sh,
           sem_a, sem_b, ssem_a, ssem_b):
        cid = lax.axis_index("c")
        sid = lax.axis_index("s")
        wid = sid * NC + cid
        base = wid * (nsc * scw)

        def zrow(k, _):
            for j in range(d // LN):
                orow_a[k, pl.ds(j * LN, LN)] = jnp.zeros((LN,), jnp.float32)
            return 0

        lax.fori_loop(0, ZR, zrow, 0)

        def zacc(ch, _):
            @pl.when(lax.rem(ch, NS) == sid)
            def _():
                pltpu.sync_copy(orow_a, acc_sh.at[pl.ds(ch * ZR, ZR)])
            return 0

        lax.fori_loop(0, nzc, zacc, 0)
        plsc.subcore_barrier()

        bufs = (grow_a, grow_b)
        sems = (sem_a, sem_b)
        orows = (orow_a, orow_b)
        ssems = (ssem_a, ssem_b)

        def schunk(si, _):
            sbase = base + si * scw
            pltpu.sync_copy(srcp_hbm.at[pl.ds(sbase, scw)], src_sb)
            pltpu.sync_copy(dstp_hbm.at[pl.ds(sbase, scw)], dst_sb)
            for hh in range(h):
                pltpu.sync_copy(attn_hbm.at[hh, pl.ds(sbase, scw)],
                                at_v.at[hh])
            cps = [None, None]
            scs = [None, None]
            cps[0] = pltpu.async_copy(
                hw_hbm.at[src_sb.at[pl.ds(0, CKC)]], bufs[0], sems[0])
            for b in range(SB):
                p = b % 2
                cps[p].wait()
                if b + 1 < SB:
                    q = (b + 1) % 2
                    cps[q] = pltpu.async_copy(
                        hw_hbm.at[src_sb.at[pl.ds((b + 1) * CKC, CKC)]],
                        bufs[q], sems[q])
                buf = bufs[p]
                orow = orows[p]
                if scs[p] is not None:
                    scs[p].wait()

                def scale(k, _):
                    kv = (jnp.broadcast_to(k, (LN,)).astype(jnp.int32)
                          + b * CKC)
                    avs = [plsc.load_gather(
                        at_v, [jnp.full((LN,), hh, jnp.int32), kv])
                        for hh in range(h)]
                    for jj in range(d // (2 * LN)):
                        ta = tb = None
                        for hh in range(h):
                            u = plsc.bitcast(
                                buf[k, pl.ds(hh * (d // 2) + jj * LN, LN)],
                                jnp.bfloat16)
                            ea, eb = plsc.unpack(
                                u, format=plsc.PackFormat.INTERLEAVED)
                            if ta is None:
                                ta = ea * avs[hh]
                                tb = eb * avs[hh]
                            else:
                                ta = ta + ea * avs[hh]
                                tb = tb + eb * avs[hh]
                        orow[k, pl.ds(jj * 2 * LN, LN)] = ta
                        orow[k, pl.ds(jj * 2 * LN + LN, LN)] = tb
                    return 0

                lax.fori_loop(0, CKC, scale, 0)
                scs[p] = pltpu.async_copy(
                    orow, acc_sh.at[dst_sb.at[pl.ds(b * CKC, CKC)]],
                    ssems[p], add=True)
            for p in range(2):
                if scs[p] is not None:
                    scs[p].wait()
            return 0

        lax.fori_loop(0, nsc, schunk, 0)

        plsc.subcore_barrier()

        def wout(ch, _):
            @pl.when(lax.rem(ch, NS) == sid)
            def _():
                pltpu.sync_copy(acc_sh.at[pl.ds(ch * ZR, ZR)],
                                out_hbm.at[cid, pl.ds(ch * ZR, ZR)])
            return 0

        lax.fori_loop(0, nzc, wout, 0)

    return kc(hw2, attn, srcp2, dstp2)


def _node_body(parts_ref, b_gat_ref, wg1_ref, bg1_ref, wg2_ref, bg2_ref,
               h2_ref, g_ref):
    p = (parts_ref[0] + parts_ref[1]) * 0.25
    h2 = _leaky(p + b_gat_ref[...], 0.1)
    h2_ref[...] = h2
    u = jnp.maximum(
        lax.dot_general(h2, wg1_ref[...], (((1,), (1,)), ((), ())),
                        preferred_element_type=jnp.float32) + bg1_ref[...],
        0.0)
    g_ref[...] = jnp.sum(u * wg2_ref[...], axis=1, keepdims=True) + bg2_ref[0, 0]


def _node_stage(parts, b_gat, wg1, bg1, wg2, bg2, n, d, bn, npad):
    nb = n // bn
    return pl.pallas_call(
        _node_body,
        grid=(nb,),
        in_specs=[
            pl.BlockSpec((NC, bn, d), lambda i: (0, i, 0)),
            pl.BlockSpec((1, d), lambda i: (0, 0)),
            pl.BlockSpec((d, d), lambda i: (0, 0)),
            pl.BlockSpec((1, d), lambda i: (0, 0)),
            pl.BlockSpec((1, d), lambda i: (0, 0)),
            pl.BlockSpec((1, 1), lambda i: (0, 0)),
        ],
        out_specs=[
            pl.BlockSpec((bn, d), lambda i: (i, 0)),
            pl.BlockSpec((bn, 1), lambda i: (i, 0)),
        ],
        out_shape=[
            jax.ShapeDtypeStruct((n, d), jnp.float32),
            jax.ShapeDtypeStruct((n, 1), jnp.float32),
        ],
    )(parts, b_gat.reshape(1, d), wg1, bg1.reshape(1, d), wg2.reshape(1, d),
      bg2.reshape(1, 1))


def _pool_body(h2_ref, g_ref, batch_ref, w_out_ref, b_out_ref, out_ref,
               *, num_graphs):
    gv = g_ref[...]
    seg = batch_ref[...]
    n = gv.shape[1]
    ids = lax.broadcasted_iota(jnp.int32, (num_graphs, n), 0)
    cmp = ids == seg
    cmpf = cmp.astype(jnp.float32)
    gb = jnp.where(cmp, jnp.broadcast_to(gv, (num_graphs, n)), -1e30)
    m = jnp.max(gb, axis=1, keepdims=True)
    mn = jnp.sum(cmpf * m, axis=0).reshape(1, n)
    e = jnp.exp(gv - mn)
    s = jnp.sum(cmpf * e, axis=1, keepdims=True)
    sn = jnp.sum(cmpf * s, axis=0).reshape(1, n)
    w = (e / sn).reshape(n, 1)
    h2w = h2_ref[...] * w
    emb = lax.dot_general(cmpf, h2w, (((1,), (0,)), ((), ())),
                          preferred_element_type=jnp.float32)
    out_ref[...] = lax.dot_general(
        emb, w_out_ref[...], (((1,), (1,)), ((), ())),
        preferred_element_type=jnp.float32) + b_out_ref[...]


def _pool(h2, g2, batch2, w_out, b_out, num_graphs, out_dim):
    return pl.pallas_call(
        functools.partial(_pool_body, num_graphs=num_graphs),
        out_shape=jax.ShapeDtypeStruct((num_graphs, out_dim), jnp.float32),
    )(h2, g2, batch2, w_out, b_out.reshape(1, out_dim))


def kernel(x, edge_index, batch, W_in, b_in, W_gat, att_src, att_dst,
           b_gat, Wg1, bg1, Wg2, bg2, W_out, b_out):
    n, d = x.shape
    h = att_src.shape[0]
    out_dim = W_out.shape[0]
    num_graphs = 64
    e = edge_index.shape[1]
    etot = e + n
    nch = -(-etot // (NW * CK))
    ep = NW * CK * nch
    bn = 1000
    npad = -(-n // ZR) * ZR

    loop = jnp.arange(n, dtype=edge_index.dtype)
    pad = jnp.zeros((ep - etot,), edge_index.dtype)
    srcp = jnp.concatenate([edge_index[0], loop, pad])
    dstp = jnp.concatenate([edge_index[1], loop, pad])

    idx = jnp.arange(d)
    grp, r = idx // 32, idx % 32
    perm = 32 * grp + jnp.where(r % 2 == 0, r // 2, 16 + r // 2)
    w_gat_p = W_gat.reshape(h, d, d)[:, perm, :].reshape(h * d, d)
    att_src_p = att_src[:, perm]
    att_dst_p = att_dst[:, perm]

    hw, asrc3, adst3 = _prep(x, W_in, b_in, w_gat_p, att_src_p, att_dst_p,
                             n, d, h, bn)
    asrc = asrc3.reshape(h, n)
    adst = adst3.reshape(h, n)

    ex, dpart = _edge_softmax_denom(asrc, adst, srcp, dstp, n, h, etot, nch)
    rinv = _rinv(dpart, n, h)
    attn = _edge_attn(rinv, ex, dstp, n, h, nch)

    hw_i32 = lax.bitcast_convert_type(
        hw.reshape(n, (h * d) // 2, 2), jnp.int32)
    parts = _edge_aggregate(hw_i32, attn, srcp, dstp, n, d, h, nch, npad)

    h2, g3 = _node_stage(parts, b_gat, Wg1, bg1, Wg2, bg2, n, d, bn, npad)
    g2 = g3.reshape(1, n)
    batch2 = batch.reshape(1, n)

    return _pool(h2, g2, batch2, W_out, b_out, num_graphs, out_dim)
